# baseline (speedup 1.0000x reference)
import functools
import math

import jax
import jax.numpy as jnp
import numpy as np
from jax import lax
from jax.experimental import pallas as pl
from jax.experimental.pallas import tpu as pltpu
from jax.experimental.pallas import tpu_sc as plsc

N_NODES = 10000
N_EDGES = 320000
F = 128
D = 4

NC, NS = 2, 16
NW = NC * NS
EB = 128
EPB = EB * NW
NB = 80
NBUF = 4
DDEPTH = 8
EPAD = NB * EPB
NPAD = 10240
RSF = NPAD // NS
RB = 512
GRID = NPAD // RB


def _bern_coeffs(d):
    thetas = []
    for i in range(d):
        beta_val = math.factorial(i) * math.factorial(d - i) / math.factorial(d + 1)
        c1 = np.zeros(i + 1)
        c1[i] = 0.5 ** i
        c2 = np.array([math.comb(d - i, j) * (-0.5) ** j for j in range(d - i + 1)])
        c = np.polynomial.polynomial.polymul(c1, c2) / (2.0 * beta_val)
        c = np.concatenate([c, np.zeros(max(0, d + 1 - c.shape[0]))])
        thetas.append([float(c[j]) for j in range(d)])
    return thetas


_COEF = _bern_coeffs(D)

_mesh = plsc.VectorSubcoreMesh(core_axis_name="c", subcore_axis_name="s")


@functools.partial(
    pl.kernel,
    out_type=jax.ShapeDtypeStruct((NC, NPAD, F), jnp.float32),
    mesh=_mesh,
    scratch_types=[
        pltpu.VMEM((NB, EB), jnp.int32),
        pltpu.VMEM((EB, F), jnp.float32),
        pltpu.VMEM_SHARED((NPAD, F), jnp.float32),
        pltpu.SemaphoreType.DMA,
    ],
)
def _sc_degree(src_hbm, ones_hbm, zeros_hbm, out_hbm, idx_v, ones_v, acc_sh, sem_s):
    c = lax.axis_index("c")
    s = lax.axis_index("s")
    r0 = s * RSF
    pltpu.sync_copy(zeros_hbm.at[pl.ds(r0, RSF)], acc_sh.at[pl.ds(r0, RSF)])
    pltpu.sync_copy(ones_hbm, ones_v)
    plsc.subcore_barrier()
    w = c * NS + s
    pltpu.sync_copy(src_hbm.at[pl.ds(w * NB, NB)], idx_v)

    def _drain1():
        pltpu.make_async_copy(zeros_hbm.at[pl.ds(0, EB)], ones_v, sem_s).wait()

    for i in range(DDEPTH):
        pltpu.async_copy(ones_v, acc_sh.at[idx_v.at[i]], sem_s, add=True)

    def body(b, carry):
        _drain1()
        pltpu.async_copy(ones_v, acc_sh.at[idx_v.at[b + DDEPTH]], sem_s, add=True)
        return carry

    lax.fori_loop(0, NB - DDEPTH, body, 0)
    for i in range(DDEPTH):
        _drain1()
    plsc.subcore_barrier()
    pltpu.sync_copy(acc_sh.at[pl.ds(r0, RSF)], out_hbm.at[c, pl.ds(r0, RSF)])


@functools.partial(
    pl.kernel,
    out_type=jax.ShapeDtypeStruct((NC, NPAD, F), jnp.float32),
    mesh=_mesh,
    scratch_types=[
        pltpu.VMEM((EB,), jnp.int32),
        pltpu.VMEM((EB,), jnp.int32),
        pltpu.VMEM((EB, F), jnp.float32),
        pltpu.VMEM_SHARED((NPAD, F), jnp.float32),
        pltpu.SemaphoreType.DMA,
    ],
)
def _sc_aggregate(xs_hbm, src_hbm, dst_hbm, zeros_hbm, out_hbm,
                  si_v, di_v, rows_v, acc_sh, sem_g):
    c = lax.axis_index("c")
    s = lax.axis_index("s")
    r0 = s * RSF
    pltpu.sync_copy(zeros_hbm.at[pl.ds(r0, RSF)], acc_sh.at[pl.ds(r0, RSF)])
    plsc.subcore_barrier()
    w = c * NS + s

    def body(b, carry):
        off = (w * NB + b) * EB
        pltpu.sync_copy(src_hbm.at[pl.ds(off, EB)], si_v)
        pltpu.sync_copy(dst_hbm.at[pl.ds(off, EB)], di_v)
        pltpu.async_copy(xs_hbm.at[si_v], rows_v, sem_g).wait()
        pltpu.sync_copy(rows_v, acc_sh.at[di_v], add=True)
        return carry

    lax.fori_loop(0, NB, body, 0)
    plsc.subcore_barrier()
    pltpu.sync_copy(acc_sh.at[pl.ds(r0, RSF)], out_hbm.at[c, pl.ds(r0, RSF)])


def _tc_pre_body(x_ref, w_ref, b_ref, degp_ref, h_ref, xs_ref, di_ref):
    deg = degp_ref[0][:, 0:1] + degp_ref[1][:, 0:1]
    di = lax.rsqrt(jnp.maximum(deg, 1.0))
    h = jnp.dot(x_ref[...], w_ref[...], preferred_element_type=jnp.float32)
    h = h + b_ref[...]
    h = jnp.where(h >= 0, h, 0.01 * h)
    h_ref[...] = h
    xs_ref[...] = h * di
    di_ref[...] = di


_tc_pre = pl.pallas_call(
    _tc_pre_body,
    grid=(GRID,),
    in_specs=[
        pl.BlockSpec((RB, F), lambda i: (i, 0)),
        pl.BlockSpec((F, F), lambda i: (0, 0)),
        pl.BlockSpec((1, F), lambda i: (0, 0)),
        pl.BlockSpec((NC, RB, F), lambda i: (0, i, 0)),
    ],
    out_specs=[
        pl.BlockSpec((RB, F), lambda i: (i, 0)),
        pl.BlockSpec((RB, F), lambda i: (i, 0)),
        pl.BlockSpec((RB, 1), lambda i: (i, 0)),
    ],
    out_shape=[
        jax.ShapeDtypeStruct((NPAD, F), jnp.float32),
        jax.ShapeDtypeStruct((NPAD, F), jnp.float32),
        jax.ShapeDtypeStruct((NPAD, 1), jnp.float32),
    ],
)


def _tc_step_body(f_ref, parts_ref, di_ref, fn_ref, xsn_ref):
    di = di_ref[...]
    agg = parts_ref[0] + parts_ref[1]
    fn = f_ref[...] - agg * di
    fn_ref[...] = fn
    xsn_ref[...] = fn * di


_tc_step = pl.pallas_call(
    _tc_step_body,
    grid=(GRID,),
    in_specs=[
        pl.BlockSpec((RB, F), lambda i: (i, 0)),
        pl.BlockSpec((NC, RB, F), lambda i: (0, i, 0)),
        pl.BlockSpec((RB, 1), lambda i: (i, 0)),
    ],
    out_specs=[
        pl.BlockSpec((RB, F), lambda i: (i, 0)),
        pl.BlockSpec((RB, F), lambda i: (i, 0)),
    ],
    out_shape=[
        jax.ShapeDtypeStruct((NPAD, F), jnp.float32),
        jax.ShapeDtypeStruct((NPAD, F), jnp.float32),
    ],
)


RBL = 400
GRIDL = N_NODES // RBL


def _tc_last_body(f0_ref, f1_ref, f2_ref, parts_ref, di_ref, out_ref):
    di = di_ref[...]
    f3 = f2_ref[...] - (parts_ref[0] + parts_ref[1]) * di
    fs = (f0_ref[...], f1_ref[...], f2_ref[...], f3)
    for i in range(D):
        acc = _COEF[i][0] * fs[0]
        for k in range(1, D):
            acc = acc + _COEF[i][k] * fs[k]
        out_ref[:, i * F:(i + 1) * F] = acc


_tc_last = pl.pallas_call(
    _tc_last_body,
    grid=(GRIDL,),
    in_specs=[
        pl.BlockSpec((RBL, F), lambda i: (i, 0)),
        pl.BlockSpec((RBL, F), lambda i: (i, 0)),
        pl.BlockSpec((RBL, F), lambda i: (i, 0)),
        pl.BlockSpec((NC, RBL, F), lambda i: (0, i, 0)),
        pl.BlockSpec((RBL, 1), lambda i: (i, 0)),
    ],
    out_specs=pl.BlockSpec((RBL, D * F), lambda i: (i, 0)),
    out_shape=jax.ShapeDtypeStruct((N_NODES, D * F), jnp.float32),
)


def kernel(x, edge_index, W, b):
    src = edge_index[0].astype(jnp.int32)
    dst = edge_index[1].astype(jnp.int32)
    pad = jnp.full((EPAD - N_EDGES,), N_NODES, jnp.int32)
    src_p = jnp.concatenate([src, pad])
    dst_p = jnp.concatenate([dst, pad])
    src_2d = src_p.reshape(NW * NB, EB)
    xp = jnp.pad(x, ((0, NPAD - N_NODES), (0, 0)))

    onesF = jnp.ones((EB, F), jnp.float32)
    zerosF = jnp.zeros((NPAD, F), jnp.float32)

    degp = _sc_degree(src_2d, onesF, zerosF)
    h, xs, dinv = _tc_pre(xp, W, b.reshape(1, F), degp)

    feats = [h]
    f = h
    for _ in range(D - 2):
        parts = _sc_aggregate(xs, src_p, dst_p, zerosF)
        f, xs = _tc_step(f, parts, dinv)
        feats.append(f)

    parts = _sc_aggregate(xs, src_p, dst_p, zerosF)
    return _tc_last(feats[0], feats[1], feats[2], parts, dinv)


# speedup vs baseline: 1.9665x; 1.9665x over previous
import functools
import math

import jax
import jax.numpy as jnp
import numpy as np
from jax import lax
from jax.experimental import pallas as pl
from jax.experimental.pallas import tpu as pltpu
from jax.experimental.pallas import tpu_sc as plsc

N_NODES = 10000
N_EDGES = 320000
F = 128
D = 4

NC, NS = 2, 16
NW = NC * NS
EB = 128
EPB = EB * NW
NB = 80
NBUF = 4
DDEPTH = 8
EPAD = NB * EPB
NPAD = 10240
RSF = NPAD // NS
RB = 512
GRID = NPAD // RB


def _bern_coeffs(d):
    thetas = []
    for i in range(d):
        beta_val = math.factorial(i) * math.factorial(d - i) / math.factorial(d + 1)
        c1 = np.zeros(i + 1)
        c1[i] = 0.5 ** i
        c2 = np.array([math.comb(d - i, j) * (-0.5) ** j for j in range(d - i + 1)])
        c = np.polynomial.polynomial.polymul(c1, c2) / (2.0 * beta_val)
        c = np.concatenate([c, np.zeros(max(0, d + 1 - c.shape[0]))])
        thetas.append([float(c[j]) for j in range(d)])
    return thetas


_COEF = _bern_coeffs(D)

_mesh = plsc.VectorSubcoreMesh(core_axis_name="c", subcore_axis_name="s")


@functools.partial(
    pl.kernel,
    out_type=jax.ShapeDtypeStruct((NC, NPAD, F), jnp.float32),
    mesh=_mesh,
    scratch_types=[
        pltpu.VMEM((NB, EB), jnp.int32),
        pltpu.VMEM((EB, F), jnp.float32),
        pltpu.VMEM_SHARED((NPAD, F), jnp.float32),
        pltpu.SemaphoreType.DMA,
    ],
)
def _sc_degree(src_hbm, ones_hbm, zeros_hbm, out_hbm, idx_v, ones_v, acc_sh, sem_s):
    c = lax.axis_index("c")
    s = lax.axis_index("s")
    r0 = s * RSF
    pltpu.sync_copy(zeros_hbm.at[pl.ds(r0, RSF)], acc_sh.at[pl.ds(r0, RSF)])
    pltpu.sync_copy(ones_hbm, ones_v)
    plsc.subcore_barrier()
    w = c * NS + s
    pltpu.sync_copy(src_hbm.at[pl.ds(w * NB, NB)], idx_v)

    def _drain1():
        pltpu.make_async_copy(zeros_hbm.at[pl.ds(0, EB)], ones_v, sem_s).wait()

    for i in range(DDEPTH):
        pltpu.async_copy(ones_v, acc_sh.at[idx_v.at[i]], sem_s, add=True)

    def body(b, carry):
        _drain1()
        pltpu.async_copy(ones_v, acc_sh.at[idx_v.at[b + DDEPTH]], sem_s, add=True)
        return carry

    lax.fori_loop(0, NB - DDEPTH, body, 0)
    for i in range(DDEPTH):
        _drain1()
    plsc.subcore_barrier()
    pltpu.sync_copy(acc_sh.at[pl.ds(r0, RSF)], out_hbm.at[c, pl.ds(r0, RSF)])


@functools.partial(
    pl.kernel,
    out_type=jax.ShapeDtypeStruct((NC, NPAD, F), jnp.float32),
    mesh=_mesh,
    scratch_types=[
        pltpu.VMEM((EB,), jnp.int32),
        pltpu.VMEM((EB,), jnp.int32),
        pltpu.VMEM((EB, F), jnp.float32),
        pltpu.VMEM_SHARED((NPAD, F), jnp.float32),
        pltpu.SemaphoreType.DMA,
    ],
)
def _sc_aggregate(xs_hbm, src_hbm, dst_hbm, zeros_hbm, out_hbm,
                  si_v, di_v, rows_v, acc_sh, sem_g):
    c = lax.axis_index("c")
    s = lax.axis_index("s")
    r0 = s * RSF
    pltpu.sync_copy(zeros_hbm.at[pl.ds(r0, RSF)], acc_sh.at[pl.ds(r0, RSF)])
    plsc.subcore_barrier()
    w = c * NS + s

    def body(b, carry):
        off = (w * NB + b) * EB
        pltpu.sync_copy(src_hbm.at[pl.ds(off, EB)], si_v)
        pltpu.sync_copy(dst_hbm.at[pl.ds(off, EB)], di_v)
        pltpu.async_copy(xs_hbm.at[si_v], rows_v, sem_g).wait()
        pltpu.sync_copy(rows_v, acc_sh.at[di_v], add=True)
        return carry

    lax.fori_loop(0, NB, body, 0)
    plsc.subcore_barrier()
    pltpu.sync_copy(acc_sh.at[pl.ds(r0, RSF)], out_hbm.at[c, pl.ds(r0, RSF)])


def _tc_pre_body(x_ref, w_ref, b_ref, degp_ref, h_ref, xs_ref, di_ref):
    deg = degp_ref[0][:, 0:1] + degp_ref[1][:, 0:1]
    di = lax.rsqrt(jnp.maximum(deg, 1.0))
    h = jnp.dot(x_ref[...], w_ref[...], preferred_element_type=jnp.float32)
    h = h + b_ref[...]
    h = jnp.where(h >= 0, h, 0.01 * h)
    h_ref[...] = h
    xs_ref[...] = h * di
    di_ref[...] = di


_tc_pre = pl.pallas_call(
    _tc_pre_body,
    grid=(GRID,),
    in_specs=[
        pl.BlockSpec((RB, F), lambda i: (i, 0)),
        pl.BlockSpec((F, F), lambda i: (0, 0)),
        pl.BlockSpec((1, F), lambda i: (0, 0)),
        pl.BlockSpec((NC, RB, F), lambda i: (0, i, 0)),
    ],
    out_specs=[
        pl.BlockSpec((RB, F), lambda i: (i, 0)),
        pl.BlockSpec((RB, F), lambda i: (i, 0)),
        pl.BlockSpec((RB, 1), lambda i: (i, 0)),
    ],
    out_shape=[
        jax.ShapeDtypeStruct((NPAD, F), jnp.float32),
        jax.ShapeDtypeStruct((NPAD, F), jnp.float32),
        jax.ShapeDtypeStruct((NPAD, 1), jnp.float32),
    ],
)


def _tc_step_body(f_ref, parts_ref, di_ref, fn_ref, xsn_ref):
    di = di_ref[...]
    agg = parts_ref[0] + parts_ref[1]
    fn = f_ref[...] - agg * di
    fn_ref[...] = fn
    xsn_ref[...] = fn * di


_tc_step = pl.pallas_call(
    _tc_step_body,
    grid=(GRID,),
    in_specs=[
        pl.BlockSpec((RB, F), lambda i: (i, 0)),
        pl.BlockSpec((NC, RB, F), lambda i: (0, i, 0)),
        pl.BlockSpec((RB, 1), lambda i: (i, 0)),
    ],
    out_specs=[
        pl.BlockSpec((RB, F), lambda i: (i, 0)),
        pl.BlockSpec((RB, F), lambda i: (i, 0)),
    ],
    out_shape=[
        jax.ShapeDtypeStruct((NPAD, F), jnp.float32),
        jax.ShapeDtypeStruct((NPAD, F), jnp.float32),
    ],
)


RBL = 400
GRIDL = N_NODES // RBL


def _tc_last_body(f0_ref, f1_ref, f2_ref, parts_ref, di_ref, out_ref):
    di = di_ref[...]
    f3 = f2_ref[...] - (parts_ref[0] + parts_ref[1]) * di
    fs = (f0_ref[...], f1_ref[...], f2_ref[...], f3)
    for i in range(D):
        acc = _COEF[i][0] * fs[0]
        for k in range(1, D):
            acc = acc + _COEF[i][k] * fs[k]
        out_ref[:, i * F:(i + 1) * F] = acc


_tc_last = pl.pallas_call(
    _tc_last_body,
    grid=(GRIDL,),
    in_specs=[
        pl.BlockSpec((RBL, F), lambda i: (i, 0)),
        pl.BlockSpec((RBL, F), lambda i: (i, 0)),
        pl.BlockSpec((RBL, F), lambda i: (i, 0)),
        pl.BlockSpec((NC, RBL, F), lambda i: (0, i, 0)),
        pl.BlockSpec((RBL, 1), lambda i: (i, 0)),
    ],
    out_specs=pl.BlockSpec((RBL, D * F), lambda i: (i, 0)),
    out_shape=jax.ShapeDtypeStruct((N_NODES, D * F), jnp.float32),
)


def kernel(x, edge_index, W, b):
    src = edge_index[0].astype(jnp.int32)
    dst = edge_index[1].astype(jnp.int32)
    pad = N_NODES + (jnp.arange(EPAD - N_EDGES, dtype=jnp.int32)
                     % (NPAD - N_NODES))
    src_p = jnp.concatenate([src, pad])
    dst_p = jnp.concatenate([dst, pad])
    src_2d = src_p.reshape(NW * NB, EB)
    xp = jnp.pad(x, ((0, NPAD - N_NODES), (0, 0)))

    onesF = jnp.ones((EB, F), jnp.float32)
    zerosF = jnp.zeros((NPAD, F), jnp.float32)

    degp = _sc_degree(src_2d, onesF, zerosF)
    h, xs, dinv = _tc_pre(xp, W, b.reshape(1, F), degp)

    feats = [h]
    f = h
    for _ in range(D - 2):
        parts = _sc_aggregate(xs, src_p, dst_p, zerosF)
        f, xs = _tc_step(f, parts, dinv)
        feats.append(f)

    parts = _sc_aggregate(xs, src_p, dst_p, zerosF)
    return _tc_last(feats[0], feats[1], feats[2], parts, dinv)
